# baseline (speedup 1.0000x reference)
import functools
import math

import jax
import jax.numpy as jnp
from jax import lax
from jax.experimental import pallas as pl
from jax.experimental.pallas import tpu as pltpu
from jax.experimental.pallas import tpu_sc as plsc

VOCAB = 1000000
D = 64
SCALE = math.sqrt(D)
NC, NS = 2, 16
NW = NC * NS
CH = 200

TOT_A = VOCAB // CH
NCH_A = -(-TOT_A // NW)


def _pipe2(nch, start_in, wait_in, proc, start_out, wait_out):
    def body(c, b, head, tail):
        wait_in(c, b)
        if not head:
            wait_out(c - 2, b)
        proc(c, b)
        start_out(c, b)
        if not tail:
            start_in(c + 2, b)

    start_in(0, 0)
    start_in(1, 1)
    body(0, 0, True, False)
    body(1, 1, True, False)
    main = nch - 4
    m, rem = divmod(main, 2)

    if m > 0:
        def outer(t, _):
            c0 = 2 + t * 2
            body(c0, 0, False, False)
            body(c0 + 1, 1, False, False)
            return 0
        lax.fori_loop(0, m, outer, 0)
    for i in range(rem):
        c = 2 + 2 * m + i
        body(c, c % 2, False, False)
    body(nch - 2, (nch - 2) % 2, False, True)
    body(nch - 1, (nch - 1) % 2, False, True)
    wait_out(nch - 2, (nch - 2) % 2)
    wait_out(nch - 1, (nch - 1) % 2)


def _emb_body(B, BPW,
              x_hbm, table_hbm, out_hbm,
              scratch_hbm,
              idx_all, u0, u1, w0, w1,
              g0, g1, o0, o1, bsem):
    nar = (u0, u1)
    wide = (w0, w1)
    gsem = (g0, g1)
    osem = (o0, o1)
    cidx = lax.axis_index("c")
    sid = lax.axis_index("s")
    wid = sid * NC + cidx
    base = wid * BPW
    nch_b = BPW // CH
    x1base = wid * nch_b

    pltpu.sync_copy(x_hbm.at[pl.ds(base, BPW)], idx_all)

    def a_off(k):
        return lax.rem(wid + k * NW, TOT_A) * CH

    def a_start_in(k, b):
        pltpu.async_copy(table_hbm.at[pl.ds(a_off(k), CH), :], nar[b],
                         gsem[b])

    def a_wait_in(k, b):
        pltpu.make_async_copy(table_hbm.at[pl.ds(a_off(k), CH), :], nar[b],
                              gsem[b]).wait()

    def a_proc(k, b):
        src, dst = nar[b], wide[b]

        @plsc.parallel_loop(0, CH, unroll=4)
        def _(i):
            for col in range(D // 16):
                sl = pl.ds(col * 16, 16)
                dst[i, sl] = src[i, sl]

    def a_start_out(k, b):
        pltpu.async_copy(wide[b], scratch_hbm.at[pl.ds(a_off(k), CH), :],
                         osem[b])

    def a_wait_out(k, b):
        pltpu.make_async_copy(wide[b], scratch_hbm.at[pl.ds(a_off(k), CH), :],
                              osem[b]).wait()

    _pipe2(NCH_A, a_start_in, a_wait_in, a_proc, a_start_out, a_wait_out)

    plsc.subcore_barrier()
    pl.semaphore_signal(bsem, 1, core_index=1 - cidx)
    pl.semaphore_wait(bsem, 1)

    def b_start_in(c, b):
        pltpu.async_copy(scratch_hbm.at[idx_all.at[pl.ds(c * CH, CH)]],
                         wide[b], gsem[b])

    def b_wait_in(c, b):
        pltpu.make_async_copy(scratch_hbm.at[idx_all.at[pl.ds(c * CH, CH)]],
                              wide[b], gsem[b]).wait()

    def b_proc(c, b):
        src, dst = wide[b], nar[b]

        @plsc.parallel_loop(0, CH, unroll=4)
        def _(i):
            for col in range(D // 16):
                sl = pl.ds(col * 16, 16)
                dst[i, sl] = src[i, sl] * SCALE

    def b_start_out(c, b):
        pltpu.async_copy(nar[b], out_hbm.at[x1base + c], osem[b])

    def b_wait_out(c, b):
        pltpu.make_async_copy(nar[b], out_hbm.at[x1base + c], osem[b]).wait()

    _pipe2(nch_b, b_start_in, b_wait_in, b_proc, b_start_out, b_wait_out)


@functools.partial(jax.jit, static_argnames=("B", "R"))
def _emb(xf, table, B, R):
    BPW = B // NW
    body = functools.partial(_emb_body, B, BPW)
    run = pl.kernel(
        body,
        out_type=jax.ShapeDtypeStruct((R, CH, D), jnp.float32),
        mesh=plsc.VectorSubcoreMesh(core_axis_name="c", subcore_axis_name="s",
                                    num_cores=NC, num_subcores=NS),
        scratch_types=[
            pltpu.MemorySpace.HBM((VOCAB, 2 * D), jnp.float32),
            pltpu.VMEM((BPW,), jnp.int32),
            pltpu.VMEM((CH, D), jnp.float32),
            pltpu.VMEM((CH, D), jnp.float32),
            pltpu.VMEM((CH, 2 * D), jnp.float32),
            pltpu.VMEM((CH, 2 * D), jnp.float32),
            pltpu.SemaphoreType.DMA,
            pltpu.SemaphoreType.DMA,
            pltpu.SemaphoreType.DMA,
            pltpu.SemaphoreType.DMA,
            pltpu.SemaphoreType.REGULAR,
        ],
    )
    return run(xf, table)


def kernel(x, table):
    R, C = x.shape
    assert C == CH and table.shape == (VOCAB, D)
    B = R * C
    xf = x.reshape(B).astype(jnp.int32)
    out = _emb(xf, table, B, R)
    return out.reshape(R, C, D)
